# baseline (speedup 1.0000x reference)
import jax
import jax.numpy as jnp
from jax import lax
from jax.experimental import pallas as pl
from jax.experimental.pallas import tpu as pltpu
from jax.experimental.pallas import tpu_sc as plsc

N_CHANNELS = 1024
N_DUP = 64
N_BATCH = 16384
LANES = 16

NUM_CORES = 2
NUM_SUBCORES = 16
NUM_WORKERS = NUM_CORES * NUM_SUBCORES

ROWS_PER_WORKER = N_BATCH // NUM_WORKERS
SCHEDULE = [32] + [40] * 12
assert sum(SCHEDULE) == ROWS_PER_WORKER
OFFSETS = [sum(SCHEDULE[:i]) for i in range(len(SCHEDULE))]
NUM_CHUNKS = len(SCHEDULE)
BUF_ROWS = max(SCHEDULE)
NBUF = 3


def _scrub(v):
    return jnp.where(v != v, jnp.float32(0.0), v)


def _mask_chunk(buf, rows, minv, maxv):
    for cg in range(N_DUP // LANES):
        c0 = cg * LANES
        mn = minv[pl.ds(c0, LANES)]
        mx = maxv[pl.ds(c0, LANES)]

        @plsc.parallel_loop(0, rows, unroll=4)
        def _rows(r, c0=c0, mn=mn, mx=mx):
            v1 = _scrub(buf[r, pl.ds(c0, LANES)])
            v2 = _scrub(buf[r, pl.ds(c0 + N_DUP, LANES)])
            in1 = (v1 >= mn) & (v1 <= mx)
            in2 = (v2 >= mn) & (v2 <= mx)
            res = jnp.where(
                in1 & in2,
                jnp.minimum(v1, v2),
                jnp.where(in2, v2, jnp.where(in1, v1, jnp.float32(0.0))),
            )
            buf[r, pl.ds(c0, LANES)] = res

    @pl.loop(N_DUP // (2 * LANES), N_CHANNELS // (2 * LANES))
    def _cols(cgp):
        c0 = cgp * (2 * LANES)
        c1 = c0 + LANES
        mn0 = minv[pl.ds(c0, LANES)]
        mx0 = maxv[pl.ds(c0, LANES)]
        mn1 = minv[pl.ds(c1, LANES)]
        mx1 = maxv[pl.ds(c1, LANES)]

        @plsc.parallel_loop(0, rows, unroll=4)
        def _rows(r, c0=c0, c1=c1, mn0=mn0, mx0=mx0, mn1=mn1, mx1=mx1):
            v0 = buf[r, pl.ds(c0, LANES)]
            v1 = buf[r, pl.ds(c1, LANES)]
            ok0 = (v0 >= mn0) & (v0 <= mx0)
            ok1 = (v1 >= mn1) & (v1 <= mx1)
            buf[r, pl.ds(c0, LANES)] = jnp.where(ok0, v0, jnp.float32(0.0))
            buf[r, pl.ds(c1, LANES)] = jnp.where(ok1, v1, jnp.float32(0.0))


def _edac_body(x_hbm, min_hbm, max_hbm, out_hbm, minv, maxv, bufs, sins, souts):
    wid = lax.axis_index("s") * NUM_CORES + lax.axis_index("c")
    base = wid * ROWS_PER_WORKER

    def chunk_dst(g):
        b = g % NBUF
        if SCHEDULE[g] == BUF_ROWS:
            return bufs[b]
        return bufs[b].at[pl.ds(0, SCHEDULE[g])]

    def start_in(g):
        b = g % NBUF
        return pltpu.async_copy(
            x_hbm.at[pl.ds(base + OFFSETS[g], SCHEDULE[g])], chunk_dst(g), sins[b]
        )

    mm0 = pltpu.async_copy(min_hbm, minv, souts[0])
    mm1 = pltpu.async_copy(max_hbm, maxv, souts[1])
    desc_in = {}
    desc_out = {}
    for g in range(min(NBUF - 1, NUM_CHUNKS)):
        desc_in[g] = start_in(g)
    mm0.wait()
    mm1.wait()

    for g in range(NUM_CHUNKS):
        b = g % NBUF
        desc_in[g].wait()
        _mask_chunk(bufs[b], SCHEDULE[g], minv, maxv)
        desc_out[g] = pltpu.async_copy(
            chunk_dst(g), out_hbm.at[pl.ds(base + OFFSETS[g], SCHEDULE[g])], souts[b]
        )
        ng = g + NBUF - 1
        if ng < NUM_CHUNKS:
            if g >= 1:
                desc_out[g - 1].wait()
            desc_in[ng] = start_in(ng)

    for g in range(max(0, NUM_CHUNKS - NBUF), NUM_CHUNKS):
        desc_out[g].wait()


def _edac_sc(x, min_val, max_val):
    mesh = plsc.VectorSubcoreMesh(
        core_axis_name="c", subcore_axis_name="s",
        num_cores=NUM_CORES, num_subcores=NUM_SUBCORES,
    )

    def body(x_hbm, min_hbm, max_hbm, out_hbm, minv, maxv, *scratch):
        bufs = list(scratch[:NBUF])
        sins = list(scratch[NBUF:2 * NBUF])
        souts = list(scratch[2 * NBUF:3 * NBUF])
        _edac_body(x_hbm, min_hbm, max_hbm, out_hbm, minv, maxv,
                   bufs, sins, souts)

    run = pl.kernel(
        body,
        out_type=jax.ShapeDtypeStruct((N_BATCH, N_CHANNELS), jnp.float32),
        mesh=mesh,
        scratch_types=(
            [pltpu.VMEM((N_CHANNELS,), jnp.float32)] * 2
            + [pltpu.VMEM((BUF_ROWS, N_CHANNELS), jnp.float32)] * NBUF
            + [pltpu.SemaphoreType.DMA] * (2 * NBUF)
        ),
    )
    return run(x, min_val, max_val)


@jax.jit
def kernel(x, min_val, max_val):
    return _edac_sc(x, min_val, max_val)


# speedup vs baseline: 1.0033x; 1.0033x over previous
import jax
import jax.numpy as jnp
from jax import lax
from jax.experimental import pallas as pl
from jax.experimental.pallas import tpu as pltpu
from jax.experimental.pallas import tpu_sc as plsc

N_CHANNELS = 1024
N_DUP = 64
N_BATCH = 16384
LANES = 16

NUM_CORES = 2
NUM_SUBCORES = 16
NUM_WORKERS = NUM_CORES * NUM_SUBCORES

ROWS_PER_WORKER = N_BATCH // NUM_WORKERS
SCHEDULE = [16] + [40] * 12 + [16]
assert sum(SCHEDULE) == ROWS_PER_WORKER
OFFSETS = [sum(SCHEDULE[:i]) for i in range(len(SCHEDULE))]
NUM_CHUNKS = len(SCHEDULE)
BUF_ROWS = max(SCHEDULE)
NBUF = 3


def _scrub(v):
    return jnp.where(v != v, jnp.float32(0.0), v)


def _mask_chunk(buf, rows, minv, maxv):
    for cg in range(N_DUP // LANES):
        c0 = cg * LANES
        mn = minv[pl.ds(c0, LANES)]
        mx = maxv[pl.ds(c0, LANES)]

        @plsc.parallel_loop(0, rows, unroll=4)
        def _rows(r, c0=c0, mn=mn, mx=mx):
            v1 = _scrub(buf[r, pl.ds(c0, LANES)])
            v2 = _scrub(buf[r, pl.ds(c0 + N_DUP, LANES)])
            in1 = (v1 >= mn) & (v1 <= mx)
            in2 = (v2 >= mn) & (v2 <= mx)
            res = jnp.where(
                in1 & in2,
                jnp.minimum(v1, v2),
                jnp.where(in2, v2, jnp.where(in1, v1, jnp.float32(0.0))),
            )
            buf[r, pl.ds(c0, LANES)] = res

    @pl.loop(N_DUP // (4 * LANES), N_CHANNELS // (4 * LANES))
    def _cols(cgp):
        cbase = cgp * (4 * LANES)
        bounds = []
        for j in range(4):
            cj = cbase + j * LANES
            bounds.append((cj, minv[pl.ds(cj, LANES)], maxv[pl.ds(cj, LANES)]))

        @plsc.parallel_loop(0, rows, unroll=2)
        def _rows(r, bounds=bounds):
            for cj, mn, mx in bounds:
                v = buf[r, pl.ds(cj, LANES)]
                ok = (v >= mn) & (v <= mx)
                buf[r, pl.ds(cj, LANES)] = jnp.where(ok, v, jnp.float32(0.0))


def _edac_body(x_hbm, min_hbm, max_hbm, out_hbm, minv, maxv, bufs, sins, souts):
    wid = lax.axis_index("s") * NUM_CORES + lax.axis_index("c")
    base = wid * ROWS_PER_WORKER

    def chunk_dst(g):
        b = g % NBUF
        if SCHEDULE[g] == BUF_ROWS:
            return bufs[b]
        return bufs[b].at[pl.ds(0, SCHEDULE[g])]

    def start_in(g):
        b = g % NBUF
        return pltpu.async_copy(
            x_hbm.at[pl.ds(base + OFFSETS[g], SCHEDULE[g])], chunk_dst(g), sins[b]
        )

    mm0 = pltpu.async_copy(min_hbm, minv, souts[0])
    mm1 = pltpu.async_copy(max_hbm, maxv, souts[1])
    desc_in = {}
    desc_out = {}
    for g in range(min(NBUF - 1, NUM_CHUNKS)):
        desc_in[g] = start_in(g)
    mm0.wait()
    mm1.wait()

    for g in range(NUM_CHUNKS):
        b = g % NBUF
        desc_in[g].wait()
        _mask_chunk(bufs[b], SCHEDULE[g], minv, maxv)
        desc_out[g] = pltpu.async_copy(
            chunk_dst(g), out_hbm.at[pl.ds(base + OFFSETS[g], SCHEDULE[g])], souts[b]
        )
        ng = g + NBUF - 1
        if ng < NUM_CHUNKS:
            if g >= 1:
                desc_out[g - 1].wait()
            desc_in[ng] = start_in(ng)

    for g in range(max(0, NUM_CHUNKS - NBUF), NUM_CHUNKS):
        desc_out[g].wait()


def _edac_sc(x, min_val, max_val):
    mesh = plsc.VectorSubcoreMesh(
        core_axis_name="c", subcore_axis_name="s",
        num_cores=NUM_CORES, num_subcores=NUM_SUBCORES,
    )

    def body(x_hbm, min_hbm, max_hbm, out_hbm, minv, maxv, *scratch):
        bufs = list(scratch[:NBUF])
        sins = list(scratch[NBUF:2 * NBUF])
        souts = list(scratch[2 * NBUF:3 * NBUF])
        _edac_body(x_hbm, min_hbm, max_hbm, out_hbm, minv, maxv,
                   bufs, sins, souts)

    run = pl.kernel(
        body,
        out_type=jax.ShapeDtypeStruct((N_BATCH, N_CHANNELS), jnp.float32),
        mesh=mesh,
        scratch_types=(
            [pltpu.VMEM((N_CHANNELS,), jnp.float32)] * 2
            + [pltpu.VMEM((BUF_ROWS, N_CHANNELS), jnp.float32)] * NBUF
            + [pltpu.SemaphoreType.DMA] * (2 * NBUF)
        ),
    )
    return run(x, min_val, max_val)


@jax.jit
def kernel(x, min_val, max_val):
    return _edac_sc(x, min_val, max_val)


# speedup vs baseline: 1.0268x; 1.0234x over previous
import jax
import jax.numpy as jnp
from jax import lax
from jax.experimental import pallas as pl
from jax.experimental.pallas import tpu as pltpu
from jax.experimental.pallas import tpu_sc as plsc

N_CHANNELS = 1024
N_DUP = 64
N_BATCH = 16384
LANES = 16

NUM_CORES = 2
NUM_SUBCORES = 16
NUM_WORKERS = NUM_CORES * NUM_SUBCORES

ROWS_PER_WORKER = N_BATCH // NUM_WORKERS
SCHEDULE = [16] + [40] * 12 + [16]
assert sum(SCHEDULE) == ROWS_PER_WORKER
OFFSETS = [sum(SCHEDULE[:i]) for i in range(len(SCHEDULE))]
NUM_CHUNKS = len(SCHEDULE)
BUF_ROWS = max(SCHEDULE)
NBUF = 3


def _scrub(v):
    return jnp.where(v != v, jnp.float32(0.0), v)


def _mask_chunk(buf, rows, minv, maxv):
    for cg in range(N_DUP // LANES):
        c0 = cg * LANES
        mn = minv[pl.ds(c0, LANES)]
        mx = maxv[pl.ds(c0, LANES)]

        @plsc.parallel_loop(0, rows, unroll=4)
        def _rows(r, c0=c0, mn=mn, mx=mx):
            v1 = _scrub(buf[r, pl.ds(c0, LANES)])
            v2 = _scrub(buf[r, pl.ds(c0 + N_DUP, LANES)])
            in1 = (v1 >= mn) & (v1 <= mx)
            in2 = (v2 >= mn) & (v2 <= mx)
            res = jnp.where(
                in1 & in2,
                jnp.minimum(v1, v2),
                jnp.where(in2, v2, jnp.where(in1, v1, jnp.float32(0.0))),
            )
            buf[r, pl.ds(c0, LANES)] = res

    @pl.loop(N_DUP // (2 * LANES), N_CHANNELS // (2 * LANES))
    def _cols(cgp):
        c0 = cgp * (2 * LANES)
        c1 = c0 + LANES
        mn0 = minv[pl.ds(c0, LANES)]
        mx0 = maxv[pl.ds(c0, LANES)]
        mn1 = minv[pl.ds(c1, LANES)]
        mx1 = maxv[pl.ds(c1, LANES)]

        @plsc.parallel_loop(0, rows, unroll=4)
        def _rows(r, c0=c0, c1=c1, mn0=mn0, mx0=mx0, mn1=mn1, mx1=mx1):
            v0 = buf[r, pl.ds(c0, LANES)]
            v1 = buf[r, pl.ds(c1, LANES)]
            ok0 = (v0 >= mn0) & (v0 <= mx0)
            ok1 = (v1 >= mn1) & (v1 <= mx1)
            buf[r, pl.ds(c0, LANES)] = jnp.where(ok0, v0, jnp.float32(0.0))
            buf[r, pl.ds(c1, LANES)] = jnp.where(ok1, v1, jnp.float32(0.0))


def _edac_body(x_hbm, min_hbm, max_hbm, out_hbm, minv, maxv, bufs, sins, souts):
    wid = lax.axis_index("s") * NUM_CORES + lax.axis_index("c")
    base = wid * ROWS_PER_WORKER

    def chunk_dst(g):
        b = g % NBUF
        if SCHEDULE[g] == BUF_ROWS:
            return bufs[b]
        return bufs[b].at[pl.ds(0, SCHEDULE[g])]

    def start_in(g):
        b = g % NBUF
        return pltpu.async_copy(
            x_hbm.at[pl.ds(base + OFFSETS[g], SCHEDULE[g])], chunk_dst(g), sins[b]
        )

    mm0 = pltpu.async_copy(min_hbm, minv, souts[0])
    mm1 = pltpu.async_copy(max_hbm, maxv, souts[1])
    desc_in = {}
    desc_out = {}
    for g in range(min(NBUF - 1, NUM_CHUNKS)):
        desc_in[g] = start_in(g)
    mm0.wait()
    mm1.wait()

    for g in range(NUM_CHUNKS):
        b = g % NBUF
        desc_in[g].wait()
        _mask_chunk(bufs[b], SCHEDULE[g], minv, maxv)
        desc_out[g] = pltpu.async_copy(
            chunk_dst(g), out_hbm.at[pl.ds(base + OFFSETS[g], SCHEDULE[g])], souts[b]
        )
        ng = g + NBUF - 1
        if ng < NUM_CHUNKS:
            if g >= 1:
                desc_out[g - 1].wait()
            desc_in[ng] = start_in(ng)

    for g in range(max(0, NUM_CHUNKS - NBUF), NUM_CHUNKS):
        desc_out[g].wait()


def _edac_sc(x, min_val, max_val):
    mesh = plsc.VectorSubcoreMesh(
        core_axis_name="c", subcore_axis_name="s",
        num_cores=NUM_CORES, num_subcores=NUM_SUBCORES,
    )

    def body(x_hbm, min_hbm, max_hbm, out_hbm, minv, maxv, *scratch):
        bufs = list(scratch[:NBUF])
        sins = list(scratch[NBUF:2 * NBUF])
        souts = list(scratch[2 * NBUF:3 * NBUF])
        _edac_body(x_hbm, min_hbm, max_hbm, out_hbm, minv, maxv,
                   bufs, sins, souts)

    run = pl.kernel(
        body,
        out_type=jax.ShapeDtypeStruct((N_BATCH, N_CHANNELS), jnp.float32),
        mesh=mesh,
        scratch_types=(
            [pltpu.VMEM((N_CHANNELS,), jnp.float32)] * 2
            + [pltpu.VMEM((BUF_ROWS, N_CHANNELS), jnp.float32)] * NBUF
            + [pltpu.SemaphoreType.DMA] * (2 * NBUF)
        ),
    )
    return run(x, min_val, max_val)


@jax.jit
def kernel(x, min_val, max_val):
    return _edac_sc(x, min_val, max_val)


# speedup vs baseline: 1.0314x; 1.0045x over previous
import jax
import jax.numpy as jnp
from jax import lax
from jax.experimental import pallas as pl
from jax.experimental.pallas import tpu as pltpu
from jax.experimental.pallas import tpu_sc as plsc

N_CHANNELS = 1024
N_DUP = 64
N_BATCH = 16384
LANES = 16

NUM_CORES = 2
NUM_SUBCORES = 16
NUM_WORKERS = NUM_CORES * NUM_SUBCORES

ROWS_PER_WORKER = N_BATCH // NUM_WORKERS
SCHEDULE = [16] + [40] * 12 + [16]
assert sum(SCHEDULE) == ROWS_PER_WORKER
OFFSETS = [sum(SCHEDULE[:i]) for i in range(len(SCHEDULE))]
NUM_CHUNKS = len(SCHEDULE)
BUF_ROWS = max(SCHEDULE)
NBUF = 3


def _scrub(v):
    return jnp.where(v != v, jnp.float32(0.0), v)


def _mask_chunk(buf, rows, minv, maxv):
    for cg in range(N_DUP // LANES):
        c0 = cg * LANES
        mn = minv[pl.ds(c0, LANES)]
        mx = maxv[pl.ds(c0, LANES)]

        @plsc.parallel_loop(0, rows, unroll=4)
        def _rows(r, c0=c0, mn=mn, mx=mx):
            v1 = _scrub(buf[r, pl.ds(c0, LANES)])
            v2 = _scrub(buf[r, pl.ds(c0 + N_DUP, LANES)])
            in1 = (v1 >= mn) & (v1 <= mx)
            in2 = (v2 >= mn) & (v2 <= mx)
            res = jnp.where(
                in1 & in2,
                jnp.minimum(v1, v2),
                jnp.where(in2, v2, jnp.where(in1, v1, jnp.float32(0.0))),
            )
            buf[r, pl.ds(c0, LANES)] = res

    @pl.loop(N_DUP // (2 * LANES), N_CHANNELS // (2 * LANES))
    def _cols(cgp):
        c0 = cgp * (2 * LANES)
        c1 = c0 + LANES
        mn0 = minv[pl.ds(c0, LANES)]
        mx0 = maxv[pl.ds(c0, LANES)]
        mn1 = minv[pl.ds(c1, LANES)]
        mx1 = maxv[pl.ds(c1, LANES)]

        @plsc.parallel_loop(0, rows, unroll=4)
        def _rows(r, c0=c0, c1=c1, mn0=mn0, mx0=mx0, mn1=mn1, mx1=mx1):
            v0 = buf[r, pl.ds(c0, LANES)]
            v1 = buf[r, pl.ds(c1, LANES)]
            ok0 = (v0 >= mn0) & (v0 <= mx0)
            ok1 = (v1 >= mn1) & (v1 <= mx1)
            buf[r, pl.ds(c0, LANES)] = jnp.where(ok0, v0, jnp.float32(0.0))
            buf[r, pl.ds(c1, LANES)] = jnp.where(ok1, v1, jnp.float32(0.0))


def _edac_body(x_hbm, min_hbm, max_hbm, out_hbm, minv, maxv, bufs, sins, souts):
    wid = lax.axis_index("s") * NUM_CORES + lax.axis_index("c")
    base = wid * ROWS_PER_WORKER

    def chunk_dst(g):
        b = g % NBUF
        if SCHEDULE[g] == BUF_ROWS:
            return bufs[b]
        return bufs[b].at[pl.ds(0, SCHEDULE[g])]

    def start_in(g):
        b = g % NBUF
        return pltpu.async_copy(
            x_hbm.at[pl.ds(base + OFFSETS[g], SCHEDULE[g])], chunk_dst(g), sins[b]
        )

    mm0 = pltpu.async_copy(min_hbm, minv, souts[0])
    mm1 = pltpu.async_copy(max_hbm, maxv, souts[1])
    desc_in = {}
    desc_out = {}
    for g in range(min(NBUF - 1, NUM_CHUNKS)):
        desc_in[g] = start_in(g)
    mm0.wait()
    mm1.wait()

    for g in range(NUM_CHUNKS):
        b = g % NBUF
        desc_in[g].wait()
        _mask_chunk(bufs[b], SCHEDULE[g], minv, maxv)
        ng = g + NBUF - 1
        if ng < NUM_CHUNKS:
            if g >= 1:
                desc_out[g - 1].wait()
            desc_in[ng] = start_in(ng)
        desc_out[g] = pltpu.async_copy(
            chunk_dst(g), out_hbm.at[pl.ds(base + OFFSETS[g], SCHEDULE[g])], souts[b]
        )

    for g in range(max(0, NUM_CHUNKS - NBUF), NUM_CHUNKS):
        desc_out[g].wait()


def _edac_sc(x, min_val, max_val):
    mesh = plsc.VectorSubcoreMesh(
        core_axis_name="c", subcore_axis_name="s",
        num_cores=NUM_CORES, num_subcores=NUM_SUBCORES,
    )

    def body(x_hbm, min_hbm, max_hbm, out_hbm, minv, maxv, *scratch):
        bufs = list(scratch[:NBUF])
        sins = list(scratch[NBUF:2 * NBUF])
        souts = list(scratch[2 * NBUF:3 * NBUF])
        _edac_body(x_hbm, min_hbm, max_hbm, out_hbm, minv, maxv,
                   bufs, sins, souts)

    run = pl.kernel(
        body,
        out_type=jax.ShapeDtypeStruct((N_BATCH, N_CHANNELS), jnp.float32),
        mesh=mesh,
        scratch_types=(
            [pltpu.VMEM((N_CHANNELS,), jnp.float32)] * 2
            + [pltpu.VMEM((BUF_ROWS, N_CHANNELS), jnp.float32)] * NBUF
            + [pltpu.SemaphoreType.DMA] * (2 * NBUF)
        ),
    )
    return run(x, min_val, max_val)


@jax.jit
def kernel(x, min_val, max_val):
    return _edac_sc(x, min_val, max_val)
